# baseline (speedup 1.0000x reference)
import functools

import jax
import jax.numpy as jnp
from jax import lax
from jax.experimental import pallas as pl
from jax.experimental.pallas import tpu as pltpu
from jax.experimental.pallas import tpu_sc as plsc

NUM_USERS = 1000000
FACTORS = 64
BATCH = 16384

NC = 2
NS = 16
NW = NC * NS
B_PER_W = BATCH // NW
CHUNK = 128
NCHUNK = B_PER_W // CHUNK
NBLK = B_PER_W // 16


def _sc_body(users_r, items_r, uemb, iemb, out_hbm,
             uidx, iidx, urows, irows, pbuf, outv, sem):
    wid = lax.axis_index("s") * NC + lax.axis_index("c")

    pltpu.sync_copy(users_r.at[pl.ds(wid * NCHUNK, NCHUNK)], uidx)
    pltpu.sync_copy(items_r.at[pl.ds(wid * NCHUNK, NCHUNK)], iidx)

    copies = []
    for j in range(NCHUNK):
        copies.append(pltpu.async_copy(
            uemb.at[uidx.at[j]], urows.at[pl.ds(j * CHUNK, CHUNK)], sem))
        copies.append(pltpu.async_copy(
            iemb.at[iidx.at[j]], irows.at[pl.ds(j * CHUNK, CHUNK)], sem))
    for c in copies:
        c.wait()

    lanes = lax.iota(jnp.int32, 16)

    def row_body(r, carry):
        s = jnp.zeros((16,), jnp.float32)
        for k in range(FACTORS // 16):
            u = urows[r, pl.ds(k * 16, 16)]
            v = irows[r, pl.ds(k * 16, 16)]
            s = s + u * v
        plsc.store_scatter(pbuf, [lanes * B_PER_W + r], s)
        return carry

    lax.fori_loop(0, B_PER_W, row_body, 0, unroll=8)

    def block(b, carry):
        acc = jnp.zeros((16,), jnp.float32)
        for l in range(16):
            acc = acc + pbuf[pl.ds(l * B_PER_W + b * 16, 16)]
        outv[pl.ds(b * 16, 16)] = acc
        return carry

    lax.fori_loop(0, NBLK, block, 0)

    pltpu.sync_copy(outv, out_hbm.at[pl.ds(wid * B_PER_W, B_PER_W)])


@jax.jit
def _mf_dot(users_r, items_r, uemb, iemb):
    mesh = plsc.VectorSubcoreMesh(core_axis_name="c", subcore_axis_name="s")
    return pl.kernel(
        _sc_body,
        mesh=mesh,
        compiler_params=pltpu.CompilerParams(
            needs_layout_passes=False, use_tc_tiling_on_sc=False),
        out_type=jax.ShapeDtypeStruct((BATCH,), jnp.float32),
        scratch_types=[
            pltpu.VMEM((NCHUNK, CHUNK), jnp.int32),
            pltpu.VMEM((NCHUNK, CHUNK), jnp.int32),
            pltpu.VMEM((B_PER_W, FACTORS), jnp.float32),
            pltpu.VMEM((B_PER_W, FACTORS), jnp.float32),
            pltpu.VMEM((16 * B_PER_W,), jnp.float32),
            pltpu.VMEM((B_PER_W,), jnp.float32),
            pltpu.SemaphoreType.DMA,
        ],
    )(users_r, items_r, uemb, iemb)


def kernel(users, items, user_emb, item_emb):
    users_r = users.astype(jnp.int32).reshape(NW * NCHUNK, CHUNK)
    items_r = items.astype(jnp.int32).reshape(NW * NCHUNK, CHUNK)
    return _mf_dot(users_r, items_r, user_emb, item_emb)


# speedup vs baseline: 2.3258x; 2.3258x over previous
import functools

import jax
import jax.numpy as jnp
from jax import lax
from jax.experimental import pallas as pl
from jax.experimental.pallas import tpu as pltpu
from jax.experimental.pallas import tpu_sc as plsc

NUM_ROWS = 1000000
FACTORS = 64
BATCH = 16384
LANES = 128

NC = 2
NS = 16
NW = NC * NS
B_PER_W = BATCH // NW
K = 4
NCHUNK = B_PER_W // K


def _sc_body(users_hbm, items_hbm, uT, iT, out_hbm,
             uidx, iidx, ubufs, ibufs, pbuf, outv, sem):
    wid = lax.axis_index("s") * NC + lax.axis_index("c")
    base = wid * B_PER_W

    pltpu.sync_copy(users_hbm.at[pl.ds(base, B_PER_W)], uidx)
    pltpu.sync_copy(items_hbm.at[pl.ds(base, B_PER_W)], iidx)

    lanes16 = lax.iota(jnp.int32, 16)
    col0 = lanes16 * B_PER_W

    def chunk(h, carry):
        j0 = h * K
        uvec = uidx[pl.ds(j0, 16)]
        ivec = iidx[pl.ds(j0, 16)]
        ustart = (uvec >> 7) << 7
        istart = (ivec >> 7) << 7
        ulane = uvec & (LANES - 1)
        ilane = ivec & (LANES - 1)

        copies = []
        for k in range(K):
            us = pl.multiple_of(ustart[k], LANES)
            its = pl.multiple_of(istart[k], LANES)
            copies.append(pltpu.async_copy(
                uT.at[:, pl.ds(us, LANES)], ubufs.at[k], sem))
            copies.append(pltpu.async_copy(
                iT.at[:, pl.ds(its, LANES)], ibufs.at[k], sem))
        for c in copies:
            c.wait()

        for k in range(K):
            lu = jnp.full((16,), ulane[k], jnp.int32)
            li = jnp.full((16,), ilane[k], jnp.int32)
            s = jnp.zeros((16,), jnp.float32)
            for q in range(FACTORS // 16):
                fvec = q * 16 + lanes16
                u = plsc.load_gather(ubufs.at[k], [fvec, lu])
                v = plsc.load_gather(ibufs.at[k], [fvec, li])
                s = s + u * v
            plsc.store_scatter(pbuf, [col0 + (j0 + k)], s)
        return carry

    lax.fori_loop(0, NCHUNK, chunk, 0)

    def block(b, carry):
        acc = jnp.zeros((16,), jnp.float32)
        for l in range(16):
            acc = acc + pbuf[pl.ds(l * B_PER_W + b * 16, 16)]
        outv[pl.ds(b * 16, 16)] = acc
        return carry

    lax.fori_loop(0, B_PER_W // 16, block, 0)

    pltpu.sync_copy(outv, out_hbm.at[pl.ds(base, B_PER_W)])


@jax.jit
def _mf_dot(users, items, uT, iT):
    mesh = plsc.VectorSubcoreMesh(core_axis_name="c", subcore_axis_name="s")
    return pl.kernel(
        _sc_body,
        mesh=mesh,
        compiler_params=pltpu.CompilerParams(needs_layout_passes=False),
        out_type=jax.ShapeDtypeStruct((BATCH,), jnp.float32),
        scratch_types=[
            pltpu.VMEM((B_PER_W,), jnp.int32),
            pltpu.VMEM((B_PER_W,), jnp.int32),
            pltpu.VMEM((K, FACTORS, LANES), jnp.float32),
            pltpu.VMEM((K, FACTORS, LANES), jnp.float32),
            pltpu.VMEM((16 * B_PER_W,), jnp.float32),
            pltpu.VMEM((B_PER_W,), jnp.float32),
            pltpu.SemaphoreType.DMA,
        ],
    )(users, items, uT, iT)


def kernel(users, items, user_emb, item_emb):
    return _mf_dot(users.astype(jnp.int32), items.astype(jnp.int32),
                   user_emb.T, item_emb.T)


# speedup vs baseline: 2.3353x; 1.0041x over previous
import functools

import jax
import jax.numpy as jnp
from jax import lax
from jax.experimental import pallas as pl
from jax.experimental.pallas import tpu as pltpu
from jax.experimental.pallas import tpu_sc as plsc

NUM_ROWS = 1000000
FACTORS = 64
BATCH = 16384
LANES = 128

NC = 2
NS = 16
NW = NC * NS
B_PER_W = BATCH // NW
K = 2


def _sc_body(users_hbm, items_hbm, uT, iT, out_hbm,
             uidx, iidx, ubufs, ibufs, pbuf, outv, semA, semB):
    wid = lax.axis_index("s") * NC + lax.axis_index("c")
    base = wid * B_PER_W

    pltpu.sync_copy(users_hbm.at[pl.ds(base, B_PER_W)], uidx)
    pltpu.sync_copy(items_hbm.at[pl.ds(base, B_PER_W)], iidx)

    lanes16 = lax.iota(jnp.int32, 16)
    col0 = lanes16 * B_PER_W
    sems = (semA, semB)

    def superchunk(s, carry):
        j0 = s * 16
        uvec = uidx[pl.ds(j0, 16)]
        ivec = iidx[pl.ds(j0, 16)]
        ustart = (uvec >> 7) << 7
        istart = (ivec >> 7) << 7
        ulane = uvec & (LANES - 1)
        ilane = ivec & (LANES - 1)

        def fire(p):
            half = p % 2
            sem = sems[half]
            copies = []
            for k in range(K):
                e = K * p + k
                us = pl.multiple_of(ustart[e], LANES)
                its = pl.multiple_of(istart[e], LANES)
                copies.append(pltpu.async_copy(
                    uT.at[:, pl.ds(us, LANES)], ubufs.at[K * half + k], sem))
                copies.append(pltpu.async_copy(
                    iT.at[:, pl.ds(its, LANES)], ibufs.at[K * half + k], sem))
            return copies

        desc = [None] * 8
        desc[0] = fire(0)
        for p in range(8):
            if p < 7:
                desc[p + 1] = fire(p + 1)
            for c in desc[p]:
                c.wait()
            half = p % 2
            for k in range(K):
                e = K * p + k
                lu = jnp.full((16,), ulane[e], jnp.int32)
                li = jnp.full((16,), ilane[e], jnp.int32)
                acc = jnp.zeros((16,), jnp.float32)
                for q in range(FACTORS // 16):
                    fvec = q * 16 + lanes16
                    u = plsc.load_gather(ubufs.at[K * half + k], [fvec, lu])
                    v = plsc.load_gather(ibufs.at[K * half + k], [fvec, li])
                    acc = acc + u * v
                plsc.store_scatter(pbuf, [col0 + (j0 + e)], acc)
        return carry

    lax.fori_loop(0, B_PER_W // 16, superchunk, 0)

    def block(b, carry):
        acc = jnp.zeros((16,), jnp.float32)
        for l in range(16):
            acc = acc + pbuf[pl.ds(l * B_PER_W + b * 16, 16)]
        outv[pl.ds(b * 16, 16)] = acc
        return carry

    lax.fori_loop(0, B_PER_W // 16, block, 0)

    pltpu.sync_copy(outv, out_hbm.at[pl.ds(base, B_PER_W)])


@jax.jit
def _mf_dot(users, items, uT, iT):
    mesh = plsc.VectorSubcoreMesh(core_axis_name="c", subcore_axis_name="s")
    return pl.kernel(
        _sc_body,
        mesh=mesh,
        compiler_params=pltpu.CompilerParams(needs_layout_passes=False),
        out_type=jax.ShapeDtypeStruct((BATCH,), jnp.float32),
        scratch_types=[
            pltpu.VMEM((B_PER_W,), jnp.int32),
            pltpu.VMEM((B_PER_W,), jnp.int32),
            pltpu.VMEM((2 * K, FACTORS, LANES), jnp.float32),
            pltpu.VMEM((2 * K, FACTORS, LANES), jnp.float32),
            pltpu.VMEM((16 * B_PER_W,), jnp.float32),
            pltpu.VMEM((B_PER_W,), jnp.float32),
            pltpu.SemaphoreType.DMA,
            pltpu.SemaphoreType.DMA,
        ],
    )(users, items, uT, iT)


def kernel(users, items, user_emb, item_emb):
    return _mf_dot(users.astype(jnp.int32), items.astype(jnp.int32),
                   user_emb.T, item_emb.T)
